# Initial kernel scaffold; baseline (speedup 1.0000x reference)
import functools

import jax
import jax.numpy as jnp
from jax import lax
from jax.experimental import pallas as pl
from jax.experimental.pallas import tpu as pltpu
from jax.experimental.pallas import tpu_sc as plsc

_NC = 2
_NS = 16
_NW = _NC * _NS

_CHUNK = 2048


def _make_gather(n: int, v: int, d: int):
    assert n % _NW == 0
    per_w = n // _NW
    assert per_w % _CHUNK == 0
    steps = per_w // _CHUNK

    mesh = plsc.VectorSubcoreMesh(core_axis_name="c", subcore_axis_name="s")

    @functools.partial(
        pl.kernel,
        out_type=jax.ShapeDtypeStruct((n, d), jnp.float32),
        mesh=mesh,
        scratch_types=[
            pltpu.VMEM((_CHUNK,), jnp.int32),
            pltpu.VMEM((_CHUNK, d), jnp.float32),
            pltpu.SemaphoreType.DMA,
        ],
    )
    def gather_kernel(idx_hbm, table_hbm, out_hbm, idx_v, rows_v, sem):
        wid = lax.axis_index("s") * _NC + lax.axis_index("c")
        base = wid * per_w

        @pl.loop(0, steps)
        def _step(g):
            off = base + g * _CHUNK
            pltpu.sync_copy(idx_hbm.at[pl.ds(off, _CHUNK)], idx_v)
            pltpu.async_copy(table_hbm.at[idx_v], rows_v, sem).wait()
            pltpu.sync_copy(rows_v, out_hbm.at[pl.ds(off, _CHUNK)])

    return gather_kernel


def kernel(farm_ids, table):
    b, h = farm_ids.shape
    v, d = table.shape
    n = b * h
    idx_flat = farm_ids.reshape(n).astype(jnp.int32)
    out_flat = _make_gather(n, v, d)(idx_flat, table)
    return out_flat.reshape(b, h, d)


# speedup vs baseline: 2.4909x; 2.4909x over previous
import functools

import jax
import jax.numpy as jnp
from jax import lax
from jax.experimental import pallas as pl
from jax.experimental.pallas import tpu as pltpu
from jax.experimental.pallas import tpu_sc as plsc

_NC = 2
_NS = 16
_NW = _NC * _NS

_CHUNK = 2048


def _make_gather(n: int, v: int, d: int):
    assert n % _NW == 0
    per_w = n // _NW
    assert per_w % _CHUNK == 0
    steps = per_w // _CHUNK

    mesh = plsc.VectorSubcoreMesh(core_axis_name="c", subcore_axis_name="s")

    @functools.partial(
        pl.kernel,
        out_type=jax.ShapeDtypeStruct((n, d), jnp.float32),
        mesh=mesh,
        compiler_params=pltpu.CompilerParams(use_tc_tiling_on_sc=False),
        scratch_types=[
            pltpu.VMEM((_CHUNK,), jnp.int32),
            pltpu.VMEM((_CHUNK, d), jnp.float32),
            pltpu.SemaphoreType.DMA,
        ],
    )
    def gather_kernel(idx_hbm, table_hbm, out_hbm, idx_v, rows_v, sem):
        wid = lax.axis_index("s") * _NC + lax.axis_index("c")
        base = wid * per_w

        @pl.loop(0, steps)
        def _step(g):
            off = base + g * _CHUNK
            pltpu.sync_copy(idx_hbm.at[pl.ds(off, _CHUNK)], idx_v)
            pltpu.async_copy(table_hbm.at[idx_v], rows_v, sem).wait()
            pltpu.sync_copy(rows_v, out_hbm.at[pl.ds(off, _CHUNK)])

    return gather_kernel


def kernel(farm_ids, table):
    b, h = farm_ids.shape
    v, d = table.shape
    n = b * h
    idx_flat = farm_ids.reshape(n).astype(jnp.int32)
    out_flat = _make_gather(n, v, d)(idx_flat, table)
    return out_flat.reshape(b, h, d)


# speedup vs baseline: 2.5320x; 1.0165x over previous
import functools

import jax
import jax.numpy as jnp
from jax import lax
from jax.experimental import pallas as pl
from jax.experimental.pallas import tpu as pltpu
from jax.experimental.pallas import tpu_sc as plsc

_NC = 2
_NS = 16
_NW = _NC * _NS

_CHUNK = 2048


def _make_gather(n: int, v: int, d: int):
    assert n % _NW == 0
    per_w = n // _NW
    assert per_w % _CHUNK == 0
    steps = per_w // _CHUNK
    assert steps >= 4 and steps % 2 == 0

    mesh = plsc.VectorSubcoreMesh(core_axis_name="c", subcore_axis_name="s")

    @functools.partial(
        pl.kernel,
        out_type=jax.ShapeDtypeStruct((n, d), jnp.float32),
        mesh=mesh,
        compiler_params=pltpu.CompilerParams(use_tc_tiling_on_sc=False),
        scratch_types=[
            pltpu.VMEM((2, _CHUNK), jnp.int32),
            pltpu.VMEM((2, _CHUNK, d), jnp.float32),
            pltpu.SemaphoreType.DMA,
            pltpu.SemaphoreType.DMA,
            pltpu.SemaphoreType.DMA,
            pltpu.SemaphoreType.DMA,
            pltpu.SemaphoreType.DMA,
            pltpu.SemaphoreType.DMA,
        ],
    )
    def gather_kernel(idx_hbm, table_hbm, out_hbm, idx_v, rows_v,
                      si0, si1, sg0, sg1, so0, so1):
        wid = lax.axis_index("s") * _NC + lax.axis_index("c")
        base = wid * per_w
        si = (si0, si1)
        sg = (sg0, sg1)
        so = (so0, so1)

        def idx_load(g, b):
            return pltpu.async_copy(
                idx_hbm.at[pl.ds(base + g * _CHUNK, _CHUNK)], idx_v.at[b], si[b])

        def gather(b):
            return pltpu.async_copy(table_hbm.at[idx_v.at[b]], rows_v.at[b], sg[b])

        def store(g, b):
            return pltpu.async_copy(
                rows_v.at[b], out_hbm.at[pl.ds(base + g * _CHUNK, _CHUNK)], so[b])

        def wait_gather(b):
            pltpu.make_async_copy(table_hbm.at[idx_v.at[b]], rows_v.at[b], sg[b]).wait()

        def wait_store(g, b):
            pltpu.make_async_copy(
                rows_v.at[b], out_hbm.at[pl.ds(base + g * _CHUNK, _CHUNK)], so[b]).wait()

        def wait_idx(g, b):
            pltpu.make_async_copy(
                idx_hbm.at[pl.ds(base + g * _CHUNK, _CHUNK)], idx_v.at[b], si[b]).wait()

        idx_load(0, 0)
        idx_load(1, 1)
        wait_idx(0, 0)
        gather(0)

        wait_gather(0)
        store(0, 0)
        wait_idx(1, 1)
        gather(1)
        idx_load(2, 0)

        @pl.loop(1, steps - 1, step=2)
        def _pair(g0):
            for off in range(2):
                g = g0 + off
                b = (1 + off) % 2
                nb = 1 - b
                wait_gather(b)
                wait_store(g - 1, nb)
                store(g, b)
                wait_idx(g + 1, nb)
                gather(nb)
                @pl.when(g + 2 < steps)
                def _():
                    idx_load(g + 2, b)

        wait_gather(1)
        wait_store(steps - 2, 0)
        store(steps - 1, 1)
        wait_store(steps - 1, 1)

    return gather_kernel


def kernel(farm_ids, table):
    b, h = farm_ids.shape
    v, d = table.shape
    n = b * h
    idx_flat = farm_ids.reshape(n).astype(jnp.int32)
    out_flat = _make_gather(n, v, d)(idx_flat, table)
    return out_flat.reshape(b, h, d)


# speedup vs baseline: 2.5697x; 1.0149x over previous
import functools

import jax
import jax.numpy as jnp
from jax import lax
from jax.experimental import pallas as pl
from jax.experimental.pallas import tpu as pltpu
from jax.experimental.pallas import tpu_sc as plsc

_NC = 2
_NS = 16
_NW = _NC * _NS

_CHUNK = 1024
_NBUF = 4


def _make_gather(n: int, v: int, d: int):
    assert n % _NW == 0
    per_w = n // _NW
    assert per_w % _CHUNK == 0
    steps = per_w // _CHUNK
    assert steps % _NBUF == 0 and steps >= 2 * _NBUF

    mesh = plsc.VectorSubcoreMesh(core_axis_name="c", subcore_axis_name="s")

    @functools.partial(
        pl.kernel,
        out_type=jax.ShapeDtypeStruct((n, d), jnp.float32),
        mesh=mesh,
        compiler_params=pltpu.CompilerParams(use_tc_tiling_on_sc=False),
        scratch_types=[
            pltpu.VMEM((_NBUF, _CHUNK), jnp.int32),
            pltpu.VMEM((_NBUF, _CHUNK, d), jnp.float32),
        ] + [pltpu.SemaphoreType.DMA] * (3 * _NBUF),
    )
    def gather_kernel(idx_hbm, table_hbm, out_hbm, idx_v, rows_v, *sems):
        si = sems[:_NBUF]
        sg = sems[_NBUF:2 * _NBUF]
        so = sems[2 * _NBUF:]
        wid = lax.axis_index("s") * _NC + lax.axis_index("c")
        base = wid * per_w

        def idx_load(g, b):
            pltpu.async_copy(
                idx_hbm.at[pl.ds(base + g * _CHUNK, _CHUNK)], idx_v.at[b], si[b])

        def wait_idx(g, b):
            pltpu.make_async_copy(
                idx_hbm.at[pl.ds(base + g * _CHUNK, _CHUNK)], idx_v.at[b], si[b]).wait()

        def gather(b):
            pltpu.async_copy(table_hbm.at[idx_v.at[b]], rows_v.at[b], sg[b])

        def wait_gather(b):
            pltpu.make_async_copy(
                table_hbm.at[idx_v.at[b]], rows_v.at[b], sg[b]).wait()

        def store(g, b):
            pltpu.async_copy(
                rows_v.at[b], out_hbm.at[pl.ds(base + g * _CHUNK, _CHUNK)], so[b])

        def wait_store(g, b):
            pltpu.make_async_copy(
                rows_v.at[b], out_hbm.at[pl.ds(base + g * _CHUNK, _CHUNK)], so[b]).wait()

        for j in range(_NBUF):
            idx_load(j, j)
        for j in range(_NBUF - 1):
            wait_idx(j, j)
            gather(j)

        @pl.loop(0, steps, step=_NBUF)
        def _ring(g0):
            for b in range(_NBUF):
                g = g0 + b
                wait_gather(b)

                @pl.when(g >= 1)
                def _():
                    wait_store(g - 1, (b - 1) % _NBUF)

                store(g, b)
                k = g + _NBUF - 1

                @pl.when(k < steps)
                def _():
                    wait_idx(k, (b - 1) % _NBUF)
                    gather((b - 1) % _NBUF)

                @pl.when(g + _NBUF < steps)
                def _():
                    idx_load(g + _NBUF, b)

        wait_store(steps - 1, (steps - 1) % _NBUF)

    return gather_kernel


def kernel(farm_ids, table):
    b, h = farm_ids.shape
    v, d = table.shape
    n = b * h
    idx_flat = farm_ids.reshape(n).astype(jnp.int32)
    out_flat = _make_gather(n, v, d)(idx_flat, table)
    return out_flat.reshape(b, h, d)


# speedup vs baseline: 4.2551x; 1.6559x over previous
import functools

import jax
import jax.numpy as jnp
from jax import lax
from jax.experimental import pallas as pl
from jax.experimental.pallas import tpu as pltpu
from jax.experimental.pallas import tpu_sc as plsc

_NC = 2
_NS = 16
_NW = _NC * _NS

_B = 16384
_H = 200
_D = 16
_HT = _H // 8
_BT = _B // 128
_UNIT = 8 * 128
_UNITS = _HT * _BT
_UPW = _UNITS // _NW
_TW = 2 * 8 * 128 * 8


def _make_kernel(v: int):
    out_words = _B * _H * _D

    mesh = plsc.VectorSubcoreMesh(core_axis_name="c", subcore_axis_name="s")

    @functools.partial(
        pl.kernel,
        out_type=jax.ShapeDtypeStruct((out_words,), jnp.float32),
        mesh=mesh,
        compiler_params=pltpu.CompilerParams(
            use_tc_tiling_on_sc=False, needs_layout_passes=False),
        scratch_types=[
            pltpu.VMEM((_UNIT,), jnp.int32),
            pltpu.VMEM((_UNIT,), jnp.int32),
            pltpu.VMEM((_UNIT, _D), jnp.float32),
            pltpu.VMEM((_UNIT, _D), jnp.float32),
            pltpu.VMEM((_TW,), jnp.float32),
            pltpu.VMEM((_TW,), jnp.float32),
        ] + [pltpu.SemaphoreType.DMA] * 6,
    )
    def gather_kernel(idx_hbm, table_hbm, out_hbm,
                      idx0, idx1, rows0, rows1, t0, t1,
                      si0, si1, sg0, sg1, so0, so1):
        idx_v = (idx0, idx1)
        rows_v = (rows0, rows1)
        t_v = (t0, t1)
        si = (si0, si1)
        sg = (sg0, sg1)
        so = (so0, so1)

        wid = lax.axis_index("s") * _NC + lax.axis_index("c")
        u0 = wid * _UPW

        ev = lax.iota(jnp.int32, 16) * 128

        def idx_off(u):
            return (u0 + u) * _UNIT

        def idx_load(u, p):
            pltpu.async_copy(
                idx_hbm.at[pl.ds(idx_off(u), _UNIT)], idx_v[p], si[p])

        def wait_idx(u, p):
            pltpu.make_async_copy(
                idx_hbm.at[pl.ds(idx_off(u), _UNIT)], idx_v[p], si[p]).wait()

        def gather(p):
            pltpu.async_copy(table_hbm.at[idx_v[p]], rows_v[p], sg[p])

        def wait_gather(p):
            pltpu.make_async_copy(table_hbm.at[idx_v[p]], rows_v[p], sg[p]).wait()

        def out_pairs(u):
            gu = u0 + u
            ht = gu // _BT
            bt = gu % _BT
            pairs = []
            for hs in range(8):
                for eb in range(2):
                    src = (hs * 2 + eb) * 1024
                    dst = (((ht * 8 + hs) * 2 + eb) * _BT + bt) * 1024
                    pairs.append((src, dst))
            return pairs

        def outs(u, p):
            for src, dst in out_pairs(u):
                pltpu.async_copy(
                    t_v[p].at[pl.ds(src, 1024)],
                    out_hbm.at[pl.ds(dst, 1024)], so[p])

        def wait_outs(u, p):
            for src, dst in out_pairs(u):
                pltpu.make_async_copy(
                    t_v[p].at[pl.ds(src, 1024)],
                    out_hbm.at[pl.ds(dst, 1024)], so[p]).wait()

        def transpose(p):
            rows = rows_v[p]
            t = t_v[p]

            @pl.loop(0, _UNIT, unroll=8)
            def _t(j):
                s = lax.shift_right_logical(j, 7) * 2048 + (j & 127)
                plsc.store_scatter(t, [ev + s], rows[j])

        idx_load(0, 0)
        idx_load(1, 1)
        wait_idx(0, 0)
        gather(0)

        @pl.loop(0, _UPW, step=2)
        def _unit(ub):
            for p in range(2):
                u = ub + p
                q = 1 - p
                wait_gather(p)

                @pl.when(u + 1 < _UPW)
                def _():
                    wait_idx(u + 1, q)
                    gather(q)

                @pl.when(u + 2 < _UPW)
                def _():
                    idx_load(u + 2, p)

                @pl.when(u >= 2)
                def _():
                    wait_outs(u - 2, p)

                transpose(p)
                outs(u, p)

        wait_outs(_UPW - 2, 0)
        wait_outs(_UPW - 1, 1)

    return gather_kernel


def kernel(farm_ids, table):
    b, h = farm_ids.shape
    v, d = table.shape
    assert (b, h, d) == (_B, _H, _D)
    idx_phys = (farm_ids.astype(jnp.int32)
                .reshape(_BT, 128, _HT, 8)
                .transpose(2, 0, 3, 1)
                .reshape(-1))
    out_flat = _make_kernel(v)(idx_phys, table)
    out = (out_flat.reshape(_H, 2, _BT, 8, 128)
           .transpose(2, 4, 0, 1, 3)
           .reshape(_B, _H, _D))
    return out


# speedup vs baseline: 5.2206x; 1.2269x over previous
import functools

import jax
import jax.numpy as jnp
from jax import lax
from jax.experimental import pallas as pl
from jax.experimental.pallas import tpu as pltpu
from jax.experimental.pallas import tpu_sc as plsc

_NC = 2
_NS = 16
_NW = _NC * _NS

_B = 16384
_H = 200
_D = 16
_HT = _H // 8
_BT = _B // 128
_UNIT = 8 * 128
_UNITS = _HT * _BT
_UPW = _UNITS // _NW
_TW = 2 * 8 * 128 * 8


def _make_kernel(v: int):
    out_words = _B * _H * _D

    mesh = plsc.VectorSubcoreMesh(core_axis_name="c", subcore_axis_name="s")

    @functools.partial(
        pl.kernel,
        out_type=jax.ShapeDtypeStruct((out_words,), jnp.float32),
        mesh=mesh,
        compiler_params=pltpu.CompilerParams(
            use_tc_tiling_on_sc=False, needs_layout_passes=False),
        scratch_types=[
            pltpu.VMEM((_UNIT,), jnp.int32),
            pltpu.VMEM((_UNIT,), jnp.int32),
            pltpu.VMEM((_UNIT, _D), jnp.float32),
            pltpu.VMEM((_UNIT, _D), jnp.float32),
            pltpu.VMEM((_TW,), jnp.float32),
            pltpu.VMEM((_TW,), jnp.float32),
        ] + [pltpu.SemaphoreType.DMA] * 6,
    )
    def gather_kernel(idx_hbm, table_hbm, out_hbm,
                      idx0, idx1, rows0, rows1, t0, t1,
                      si0, si1, sg0, sg1, so0, so1):
        idx_v = (idx0, idx1)
        rows_v = (rows0, rows1)
        t_v = (t0, t1)
        si = (si0, si1)
        sg = (sg0, sg1)
        so = (so0, so1)

        wid = lax.axis_index("s") * _NC + lax.axis_index("c")
        u0 = wid * _UPW

        ev = lax.iota(jnp.int32, 16) * 128

        def idx_off(u):
            return (u0 + u) * _UNIT

        def idx_load(u, p):
            pltpu.async_copy(
                idx_hbm.at[pl.ds(idx_off(u), _UNIT)], idx_v[p], si[p])

        def wait_idx(u, p):
            pltpu.make_async_copy(
                idx_hbm.at[pl.ds(idx_off(u), _UNIT)], idx_v[p], si[p]).wait()

        def gather(p):
            pltpu.async_copy(table_hbm.at[idx_v[p]], rows_v[p], sg[p])

        def wait_gather(p):
            pltpu.make_async_copy(table_hbm.at[idx_v[p]], rows_v[p], sg[p]).wait()

        def out_pairs(u):
            gu = u0 + u
            ht = gu // _BT
            bt = gu % _BT
            pairs = []
            for hs in range(8):
                for eb in range(2):
                    src = (hs * 2 + eb) * 1024
                    dst = (((ht * 8 + hs) * 2 + eb) * _BT + bt) * 1024
                    pairs.append((src, dst))
            return pairs

        def outs(u, p):
            for src, dst in out_pairs(u):
                pltpu.async_copy(
                    t_v[p].at[pl.ds(src, 1024)],
                    out_hbm.at[pl.ds(dst, 1024)], so[p])

        def wait_outs(u, p):
            for src, dst in out_pairs(u):
                pltpu.make_async_copy(
                    t_v[p].at[pl.ds(src, 1024)],
                    out_hbm.at[pl.ds(dst, 1024)], so[p]).wait()

        def transpose(p):
            rows = rows_v[p]
            t = t_v[p]

            @plsc.parallel_loop(0, _UNIT, unroll=8)
            def _t(j):
                s = lax.shift_right_logical(j, 7) * 2048 + (j & 127)
                plsc.store_scatter(t, [ev + s], rows[j])

        idx_load(0, 0)
        idx_load(1, 1)
        wait_idx(0, 0)
        gather(0)

        @pl.loop(0, _UPW, step=2)
        def _unit(ub):
            for p in range(2):
                u = ub + p
                q = 1 - p
                wait_gather(p)

                @pl.when(u + 1 < _UPW)
                def _():
                    wait_idx(u + 1, q)
                    gather(q)

                @pl.when(u + 2 < _UPW)
                def _():
                    idx_load(u + 2, p)

                @pl.when(u >= 2)
                def _():
                    wait_outs(u - 2, p)

                transpose(p)
                outs(u, p)

        wait_outs(_UPW - 2, 0)
        wait_outs(_UPW - 1, 1)

    return gather_kernel


def kernel(farm_ids, table):
    b, h = farm_ids.shape
    v, d = table.shape
    assert (b, h, d) == (_B, _H, _D)
    idx_phys = (farm_ids.astype(jnp.int32)
                .reshape(_BT, 128, _HT, 8)
                .transpose(2, 0, 3, 1)
                .reshape(-1))
    out_flat = _make_kernel(v)(idx_phys, table)
    out = (out_flat.reshape(_H, 2, _BT, 8, 128)
           .transpose(2, 4, 0, 1, 3)
           .reshape(_B, _H, _D))
    return out


# speedup vs baseline: 9.1224x; 1.7474x over previous
import functools

import jax
import jax.numpy as jnp
from jax import lax
from jax.experimental import pallas as pl
from jax.experimental.pallas import tpu as pltpu
from jax.experimental.pallas import tpu_sc as plsc

_NC = 2
_NS = 16
_NW = _NC * _NS

_B = 16384
_H = 200
_D = 16
_HT = _H // 8
_BT = _B // 128
_UNIT = 8 * 128
_UNITS = _HT * _BT
_UPW = _UNITS // _NW
_PITCH = 129


def _make_kernel(v: int):
    out_words = _B * _H * _D

    mesh = plsc.VectorSubcoreMesh(core_axis_name="c", subcore_axis_name="s")

    @functools.partial(
        pl.kernel,
        out_type=jax.ShapeDtypeStruct((_H, 2, _BT, 8, 128), jnp.float32),
        mesh=mesh,
        compiler_params=pltpu.CompilerParams(
            use_tc_tiling_on_sc=False, needs_layout_passes=False),
        scratch_types=[
            pltpu.VMEM((_UNIT,), jnp.int32),
            pltpu.VMEM((_UNIT,), jnp.int32),
            pltpu.VMEM((_UNIT, _D), jnp.float32),
            pltpu.VMEM((_UNIT, _D), jnp.float32),
            pltpu.VMEM((8, _D, _PITCH), jnp.float32),
            pltpu.VMEM((8, _D, _PITCH), jnp.float32),
        ] + [pltpu.SemaphoreType.DMA] * 6,
    )
    def gather_kernel(idx_hbm, table_hbm, out_hbm,
                      idx0, idx1, rows0, rows1, t0, t1,
                      si0, si1, sg0, sg1, so0, so1):
        idx_v = (idx0, idx1)
        rows_v = (rows0, rows1)
        t_v = (t0, t1)
        si = (si0, si1)
        sg = (sg0, sg1)
        so = (so0, so1)

        wid = lax.axis_index("s") * _NC + lax.axis_index("c")
        u0 = wid * _UPW

        ev = lax.iota(jnp.int32, 16)

        def idx_off(u):
            return (u0 + u) * _UNIT

        def idx_load(u, p):
            pltpu.async_copy(
                idx_hbm.at[pl.ds(idx_off(u), _UNIT)], idx_v[p], si[p])

        def wait_idx(u, p):
            pltpu.make_async_copy(
                idx_hbm.at[pl.ds(idx_off(u), _UNIT)], idx_v[p], si[p]).wait()

        def gather(p):
            pltpu.async_copy(table_hbm.at[idx_v[p]], rows_v[p], sg[p])

        def wait_gather(p):
            pltpu.make_async_copy(table_hbm.at[idx_v[p]], rows_v[p], sg[p]).wait()

        def out_pairs(u, p):
            gu = u0 + u
            ht = gu // _BT
            bt = gu % _BT
            pairs = []
            for hs in range(8):
                for eb in range(2):
                    src = t_v[p].at[hs, pl.ds(eb * 8, 8), pl.ds(0, 128)]
                    dst = out_hbm.at[ht * 8 + hs, eb, bt]
                    pairs.append((src, dst))
            return pairs

        def outs(u, p):
            for src, dst in out_pairs(u, p):
                pltpu.async_copy(src, dst, so[p])

        def wait_outs(u, p):
            for src, dst in out_pairs(u, p):
                pltpu.make_async_copy(src, dst, so[p]).wait()

        def transpose(p):
            rows = rows_v[p]
            t = t_v[p]

            @plsc.parallel_loop(0, _UNIT, unroll=8)
            def _t(j):
                hs = lax.shift_right_logical(j, 7)
                bl = j & 127
                plsc.store_scatter(
                    t, [jnp.full((16,), hs, jnp.int32), ev,
                        jnp.full((16,), bl, jnp.int32)], rows[j])

        idx_load(0, 0)
        idx_load(1, 1)
        wait_idx(0, 0)
        gather(0)

        @pl.loop(0, _UPW, step=2)
        def _unit(ub):
            for p in range(2):
                u = ub + p
                q = 1 - p
                wait_gather(p)

                @pl.when(u + 1 < _UPW)
                def _():
                    wait_idx(u + 1, q)
                    gather(q)

                @pl.when(u + 2 < _UPW)
                def _():
                    idx_load(u + 2, p)

                @pl.when(u >= 2)
                def _():
                    wait_outs(u - 2, p)

                transpose(p)
                outs(u, p)

        wait_outs(_UPW - 2, 0)
        wait_outs(_UPW - 1, 1)

    return gather_kernel


def kernel(farm_ids, table):
    b, h = farm_ids.shape
    v, d = table.shape
    assert (b, h, d) == (_B, _H, _D)
    idx_phys = (farm_ids.astype(jnp.int32)
                .reshape(_BT, 128, _HT, 8)
                .transpose(2, 0, 3, 1)
                .reshape(-1))
    out5 = _make_kernel(v)(idx_phys, table)
    return out5.transpose(2, 4, 0, 1, 3).reshape(_B, _H, _D)
